# baseline (speedup 1.0000x reference)
import functools

import jax
import jax.numpy as jnp
from jax import lax
from jax.experimental import pallas as pl
from jax.experimental.pallas import tpu as pltpu
from jax.experimental.pallas import tpu_sc as plsc

B, LS, DM = 4096, 200, 64
TOT = B * LS
NC, NS = 2, 16
NW = NC * NS
PER_W = TOT // NW
CHUNK = 512
NCHUNK = PER_W // CHUNK

_mesh = plsc.VectorSubcoreMesh(core_axis_name="c", subcore_axis_name="s")


@functools.partial(
    pl.kernel,
    out_type=jax.ShapeDtypeStruct((TOT, DM), jnp.float32),
    mesh=_mesh,
    scratch_types=[
        pltpu.VMEM((PER_W,), jnp.int32),
        pltpu.VMEM((CHUNK, DM), jnp.float32),
        pltpu.SemaphoreType.DMA,
    ],
    compiler_params=pltpu.CompilerParams(use_tc_tiling_on_sc=False),
)
def _sc_gather(idx_hbm, table_hbm, out_hbm, idx_v, rows_v, gsem):
    wid = lax.axis_index("s") * NC + lax.axis_index("c")
    base = wid * PER_W
    pltpu.sync_copy(idx_hbm.at[pl.ds(base, PER_W)], idx_v)

    def body(i, carry):
        off = i * CHUNK
        pltpu.async_copy(
            table_hbm.at[idx_v.at[pl.ds(off, CHUNK)]], rows_v, gsem
        ).wait()
        pltpu.sync_copy(rows_v, out_hbm.at[pl.ds(base + off, CHUNK)])
        return carry

    lax.fori_loop(0, NCHUNK, body, 0)


def kernel(source, table):
    idx = source.reshape(TOT).astype(jnp.int32)
    out = _sc_gather(idx, table)
    return out.reshape(B, LS, DM)


# speedup vs baseline: 1.0264x; 1.0264x over previous
import functools

import jax
import jax.numpy as jnp
from jax import lax
from jax.experimental import pallas as pl
from jax.experimental.pallas import tpu as pltpu
from jax.experimental.pallas import tpu_sc as plsc

B, LS, DM = 4096, 200, 64
TOT = B * LS
NC, NS = 2, 16
NW = NC * NS
PER_W = TOT // NW
CHUNK = 512
NCHUNK = PER_W // CHUNK

_mesh = plsc.VectorSubcoreMesh(core_axis_name="c", subcore_axis_name="s")


@functools.partial(
    pl.kernel,
    out_type=jax.ShapeDtypeStruct((TOT, DM), jnp.float32),
    mesh=_mesh,
    scratch_types=[
        pltpu.VMEM((PER_W,), jnp.int32),
        pltpu.VMEM((CHUNK, DM), jnp.float32),
        pltpu.SemaphoreType.DMA,
    ],
    compiler_params=pltpu.CompilerParams(use_tc_tiling_on_sc=False),
)
def _sc_gather(idx_hbm, table_hbm, out_hbm, idx_v, rows_v, gsem):
    wid = lax.axis_index("s") * NC + lax.axis_index("c")
    base = wid * PER_W
    pltpu.sync_copy(idx_hbm.at[pl.ds(base, PER_W)], idx_v)

    def body(i, carry):
        off = i * CHUNK
        pltpu.async_copy(
            table_hbm.at[idx_v.at[pl.ds(off, CHUNK)]], rows_v, gsem
        ).wait()
        pltpu.sync_copy(rows_v, out_hbm.at[pl.ds(base + off, CHUNK)])
        return carry

    lax.fori_loop(0, NCHUNK, body, 0)


def kernel(source, table):
    idx = source.T.reshape(TOT).astype(jnp.int32)
    out = _sc_gather(idx, table)
    return out.reshape(LS, B, DM).transpose(1, 0, 2)


# speedup vs baseline: 1.2650x; 1.2325x over previous
import functools

import jax
import jax.numpy as jnp
from jax import lax
from jax.experimental import pallas as pl
from jax.experimental.pallas import tpu as pltpu
from jax.experimental.pallas import tpu_sc as plsc

B, LS, DM = 4096, 200, 64
DP = 128
TOT = B * LS
NC, NS = 2, 16
NW = NC * NS
PER_W = TOT // NW
CHUNK = 512
NCHUNK = PER_W // CHUNK

_mesh = plsc.VectorSubcoreMesh(core_axis_name="c", subcore_axis_name="s")


@functools.partial(
    pl.kernel,
    out_type=jax.ShapeDtypeStruct((TOT, DP), jnp.float32),
    mesh=_mesh,
    scratch_types=[
        pltpu.VMEM((PER_W,), jnp.int32),
        pltpu.VMEM((CHUNK, DP), jnp.float32),
        pltpu.SemaphoreType.DMA,
    ],
)
def _sc_gather(idx_hbm, table_hbm, out_hbm, idx_v, rows_v, gsem):
    wid = lax.axis_index("s") * NC + lax.axis_index("c")
    base = wid * PER_W
    pltpu.sync_copy(idx_hbm.at[pl.ds(base, PER_W)], idx_v)

    def body(i, carry):
        off = i * CHUNK
        pltpu.async_copy(
            table_hbm.at[idx_v.at[pl.ds(off, CHUNK)]], rows_v, gsem
        ).wait()
        pltpu.sync_copy(rows_v, out_hbm.at[pl.ds(base + off, CHUNK)])
        return carry

    lax.fori_loop(0, NCHUNK, body, 0)


def kernel(source, table):
    idx = source.T.reshape(TOT).astype(jnp.int32)
    tpad = jnp.pad(table, ((0, 0), (0, DP - DM)))
    out = _sc_gather(idx, tpad)
    return out.reshape(LS, B, DP)[:, :, :DM].transpose(1, 0, 2)
